# baseline (speedup 1.0000x reference)
import functools

import jax
import jax.numpy as jnp
from jax import lax
from jax.experimental import pallas as pl
from jax.experimental.pallas import tpu as pltpu
from jax.experimental.pallas import tpu_sc as plsc

H = 128
NC = 2
NS = 16
NW = NC * NS
CHUNK = 64
FGROUPS = H // 16

def _mesh():
  return plsc.VectorSubcoreMesh(
      core_axis_name="c", subcore_axis_name="s", num_cores=NC, num_subcores=NS)


def _zero_rows(rows_v, n_rows):
  def body(i, _):
    for cg in range(FGROUPS):
      rows_v[i, pl.ds(cg * 16, 16)] = jnp.zeros((16,), jnp.float32)
    return 0
  lax.fori_loop(0, n_rows, body, 0)


NR = 4


def _make_agg(n_pad, e_pad):
  ept = e_pad // NW
  rpt = n_pad // NS
  n_steps = ept // CHUNK

  def body(hp_hbm, pk_hbm, out_hbm, pkb, rows, acc_sh, isem, gsem, ssem):
    c = lax.axis_index("c")
    s = lax.axis_index("s")
    wid = c * NS + s

    _zero_rows(rows.at[0], CHUNK)
    base_row = s * rpt
    for k in range(rpt // CHUNK):
      pltpu.async_copy(
          rows.at[0], acc_sh.at[pl.ds(base_row + k * CHUNK, CHUNK)], ssem.at[0])
    for k in range(rpt // CHUNK):
      pltpu.make_async_copy(
          rows.at[0], acc_sh.at[pl.ds(base_row + k * CHUNK, CHUNK)], ssem.at[0]).wait()
    plsc.subcore_barrier()

    def scale(b):
      rr = rows.at[b]

      def scale_body(g, _):
        w16 = lax.bitcast_convert_type(pkb[b, 2, pl.ds(g * 16, 16)], jnp.float32)
        for i in range(16):
          e = g * 16 + i
          w = w16[i]
          for cg in range(FGROUPS):
            sl = pl.ds(cg * 16, 16)
            rr[e, sl] = rr[e, sl] * w
        return 0
      lax.fori_loop(0, CHUNK // 16, scale_body, 0)

    def start_idx(j, b):
      pltpu.async_copy(pk_hbm.at[wid, j], pkb.at[b], isem.at[b])

    def wait_idx(j, b):
      pltpu.make_async_copy(pk_hbm.at[wid, j], pkb.at[b], isem.at[b]).wait()

    def start_gather(b):
      pltpu.async_copy(hp_hbm.at[pkb.at[b, 0]], rows.at[b], gsem.at[b])

    def wait_gather(b):
      pltpu.make_async_copy(
          hp_hbm.at[pkb.at[b, 0]], rows.at[b], gsem.at[b]).wait()

    def start_scatter(b):
      pltpu.async_copy(
          rows.at[b], acc_sh.at[pkb.at[b, 1]], ssem.at[b], add=True)

    def wait_scatter(b):
      pltpu.make_async_copy(
          rows.at[b], acc_sh.at[pkb.at[b, 1]], ssem.at[b]).wait()

    for b in range(2):
      start_idx(b, b)
    for b in range(2):
      wait_idx(b, b)
      start_gather(b)

    def group_body(g, _):
      for k in range(NR):
        j = g * NR + k
        kn = (k + 2) % NR
        @pl.when(j >= 2)
        def _():
          wait_scatter(kn)
        @pl.when(j + 2 < n_steps)
        def _():
          start_idx(j + 2, kn)
        wait_gather(k)
        scale(k)
        start_scatter(k)
        @pl.when(j + 2 < n_steps)
        def _():
          wait_idx(j + 2, kn)
          start_gather(kn)
      return 0

    lax.fori_loop(0, n_steps // NR, group_body, 0)
    for b in range(2):
      j = n_steps - 2 + b
      wait_scatter(j % NR)
    plsc.subcore_barrier()

    for k in range(rpt // CHUNK):
      r0 = base_row + k * CHUNK
      pltpu.async_copy(
          acc_sh.at[pl.ds(r0, CHUNK)], out_hbm.at[c, pl.ds(r0, CHUNK)], ssem.at[0])
    for k in range(rpt // CHUNK):
      r0 = base_row + k * CHUNK
      pltpu.make_async_copy(
          acc_sh.at[pl.ds(r0, CHUNK)], out_hbm.at[c, pl.ds(r0, CHUNK)], ssem.at[0]).wait()

  return pl.kernel(
      body,
      out_type=jax.ShapeDtypeStruct((NC, n_pad, H), jnp.float32),
      mesh=_mesh(),
      scratch_types=[
          pltpu.VMEM((NR, 3, CHUNK), jnp.int32),
          pltpu.VMEM((NR, CHUNK, H), jnp.float32),
          pltpu.VMEM_SHARED((n_pad, H), jnp.float32),
          pltpu.SemaphoreType.DMA((NR,)),
          pltpu.SemaphoreType.DMA((NR,)),
          pltpu.SemaphoreType.DMA((NR,)),
      ],
  )


def _make_deg(n_pad, e_pad):
  ept = e_pad // NW
  rpt = n_pad // NS
  n_steps = ept // CHUNK

  def body(dst_hbm, ew_hbm, out_hbm, dst2d, ew2d, zero_v, deg_sh, sem):
    c = lax.axis_index("c")
    s = lax.axis_index("s")
    wid = c * NS + s

    pltpu.sync_copy(dst_hbm.at[wid], dst2d)
    pltpu.sync_copy(ew_hbm.at[wid], ew2d)

    for g in range(CHUNK // 16):
      zero_v[pl.ds(g * 16, 16)] = jnp.zeros((16,), jnp.float32)
    base_row = s * rpt
    for k in range(rpt // CHUNK):
      pltpu.sync_copy(zero_v, deg_sh.at[pl.ds(base_row + k * CHUNK, CHUNK)])
    plsc.subcore_barrier()

    def fire(j, _):
      pltpu.async_copy(ew2d.at[j], deg_sh.at[dst2d.at[j]], sem, add=True)
      return 0
    lax.fori_loop(0, n_steps, fire, 0)

    def drain(j, _):
      pltpu.make_async_copy(ew2d.at[j], deg_sh.at[dst2d.at[j]], sem).wait()
      return 0
    lax.fori_loop(0, n_steps, drain, 0)
    plsc.subcore_barrier()

    for k in range(rpt // CHUNK):
      r0 = base_row + k * CHUNK
      pltpu.sync_copy(deg_sh.at[pl.ds(r0, CHUNK)], zero_v)
      pltpu.sync_copy(zero_v, out_hbm.at[c, pl.ds(r0, CHUNK)])

  return pl.kernel(
      body,
      out_type=jax.ShapeDtypeStruct((NC, n_pad), jnp.float32),
      mesh=_mesh(),
      scratch_types=[
          pltpu.VMEM((n_steps, CHUNK), jnp.int32),
          pltpu.VMEM((n_steps, CHUNK), jnp.float32),
          pltpu.VMEM((CHUNK,), jnp.float32),
          pltpu.VMEM_SHARED((n_pad,), jnp.float32),
          pltpu.SemaphoreType.DMA,
      ],
  )


_BM = 256


def _mm1_body(x_ref, w_ref, degp_ref, h_ref, hp_ref, dis_ref, dinv_ref):
  d = degp_ref[0] + degp_ref[1] + 1.0
  dinv = 1.0 / d
  dis = jnp.sqrt(dinv)
  dinv_ref[...] = dinv
  dis_ref[...] = dis
  h = jnp.dot(x_ref[...], w_ref[...], preferred_element_type=jnp.float32)
  h_ref[...] = h
  hp_ref[...] = h * dis


def _comb_mm_body(pa_ref, pb_ref, h_ref, dis_ref, dinv_ref, b_ref, w_ref,
                  out_ref, hn_ref, hpn_ref):
  o = jnp.maximum(
      dis_ref[...] * (pa_ref[...] + pb_ref[...])
      + dinv_ref[...] * h_ref[...] + b_ref[...], 0.0)
  out_ref[...] = o
  hn = jnp.dot(o, w_ref[...], preferred_element_type=jnp.float32)
  hn_ref[...] = hn
  hpn_ref[...] = hn * dis_ref[...]


def _comb_body(pa_ref, pb_ref, h_ref, dis_ref, dinv_ref, b_ref, out_ref):
  out_ref[...] = jnp.maximum(
      dis_ref[...] * (pa_ref[...] + pb_ref[...])
      + dinv_ref[...] * h_ref[...] + b_ref[...], 0.0)


def _row_specs(n_pad):
  blk = lambda i: (i, 0)
  full = lambda i: (0, 0)
  mat = pl.BlockSpec((_BM, H), blk)
  col = pl.BlockSpec((_BM, 1), blk)
  w = pl.BlockSpec((H, H), full)
  bias = pl.BlockSpec((1, H), full)
  return mat, col, w, bias, n_pad // _BM


def _mm1(n_pad, x, w1, deg_parts):
  mat, col, w, _, g = _row_specs(n_pad)
  degspec = pl.BlockSpec((2, _BM, 1), lambda i: (0, i, 0))
  return pl.pallas_call(
      _mm1_body,
      grid=(g,),
      in_specs=[mat, w, degspec],
      out_specs=[mat, mat, col, col],
      out_shape=[jax.ShapeDtypeStruct((n_pad, H), jnp.float32),
                 jax.ShapeDtypeStruct((n_pad, H), jnp.float32),
                 jax.ShapeDtypeStruct((n_pad, 1), jnp.float32),
                 jax.ShapeDtypeStruct((n_pad, 1), jnp.float32)],
  )(x, w1, deg_parts.reshape(2, n_pad, 1))


def _comb_mm(n_pad, pa, pb, h, dis_col, dinv_col, b_row, w_next):
  mat, col, w, bias, g = _row_specs(n_pad)
  return pl.pallas_call(
      _comb_mm_body,
      grid=(g,),
      in_specs=[mat, mat, mat, col, col, bias, w],
      out_specs=[mat, mat, mat],
      out_shape=[jax.ShapeDtypeStruct((n_pad, H), jnp.float32)] * 3,
  )(pa, pb, h, dis_col, dinv_col, b_row, w_next)


def _comb(n_pad, pa, pb, h, dis_col, dinv_col, b_row):
  mat, col, _, bias, g = _row_specs(n_pad)
  return pl.pallas_call(
      _comb_body,
      grid=(g,),
      in_specs=[mat, mat, mat, col, col, bias],
      out_specs=mat,
      out_shape=jax.ShapeDtypeStruct((n_pad, H), jnp.float32),
  )(pa, pb, h, dis_col, dinv_col, b_row)


def kernel(x, edge_index, edge_weights, W1, b1, W2, b2, W3, b3):
  n = x.shape[0]
  e = edge_index.shape[1]
  n_pad = ((n + NS * CHUNK - 1) // (NS * CHUNK)) * (NS * CHUNK)
  e_align = NW * CHUNK * NR
  e_pad = ((e + e_align - 1) // e_align) * e_align

  src = edge_index[0].astype(jnp.int32)
  dst = edge_index[1].astype(jnp.int32)
  ew = edge_weights.astype(jnp.float32)
  pad = e_pad - e
  if pad:
    pad_idx = jnp.arange(pad, dtype=jnp.int32) % n
    src = jnp.concatenate([src, pad_idx])
    dst = jnp.concatenate([dst, pad_idx])
    ew = jnp.concatenate([ew, jnp.zeros((pad,), jnp.float32)])
  x_p = jnp.pad(x, ((0, n_pad - n), (0, 0)))

  steps = e_pad // (NW * CHUNK)
  src = src.reshape(NW, steps, CHUNK)
  dst = dst.reshape(NW, steps, CHUNK)
  ew = ew.reshape(NW, steps, CHUNK)
  pk = jnp.stack([src, dst, lax.bitcast_convert_type(ew, jnp.int32)], axis=2)

  agg = _make_agg(n_pad, e_pad)
  deg_parts = _make_deg(n_pad, e_pad)(dst, ew)

  b1r, b2r, b3r = (b.reshape(1, H) for b in (b1, b2, b3))

  h1, hp1, dis_col, dinv_col = _mm1(n_pad, x_p, W1, deg_parts)
  p1 = agg(hp1, pk)
  o1, h2, hp2 = _comb_mm(n_pad, p1[0], p1[1], h1, dis_col, dinv_col, b1r, W2)
  p2 = agg(hp2, pk)
  o2, h3, hp3 = _comb_mm(n_pad, p2[0], p2[1], h2, dis_col, dinv_col, b2r, W3)
  p3 = agg(hp3, pk)
  o3 = _comb(n_pad, p3[0], p3[1], h3, dis_col, dinv_col, b3r)

  return jnp.concatenate([o1[:n], o2[:n], o3[:n]], axis=1)


# speedup vs baseline: 1.0791x; 1.0791x over previous
import functools

import jax
import jax.numpy as jnp
from jax import lax
from jax.experimental import pallas as pl
from jax.experimental.pallas import tpu as pltpu
from jax.experimental.pallas import tpu_sc as plsc

H = 128
NC = 2
NS = 16
NW = NC * NS
CHUNK = 64
FGROUPS = H // 16

def _mesh():
  return plsc.VectorSubcoreMesh(
      core_axis_name="c", subcore_axis_name="s", num_cores=NC, num_subcores=NS)


def _zero_rows(rows_v, n_rows):
  def body(i, _):
    for cg in range(FGROUPS):
      rows_v[i, pl.ds(cg * 16, 16)] = jnp.zeros((16,), jnp.float32)
    return 0
  lax.fori_loop(0, n_rows, body, 0)


NR = 5


def _make_agg(n_pad, e_pad):
  ept = e_pad // NW
  rpt = n_pad // NS
  n_steps = ept // CHUNK

  def body(hp_hbm, pk_hbm, out_hbm, pkb, rows, acc_sh, isem, gsem, ssem):
    c = lax.axis_index("c")
    s = lax.axis_index("s")
    wid = c * NS + s

    _zero_rows(rows.at[0], CHUNK)
    base_row = s * rpt
    for k in range(rpt // CHUNK):
      pltpu.async_copy(
          rows.at[0], acc_sh.at[pl.ds(base_row + k * CHUNK, CHUNK)], ssem.at[0])
    for k in range(rpt // CHUNK):
      pltpu.make_async_copy(
          rows.at[0], acc_sh.at[pl.ds(base_row + k * CHUNK, CHUNK)], ssem.at[0]).wait()
    plsc.subcore_barrier()

    def scale(b):
      rr = rows.at[b]

      def scale_body(g, _):
        w16 = lax.bitcast_convert_type(pkb[b, 2, pl.ds(g * 16, 16)], jnp.float32)
        for i in range(16):
          e = g * 16 + i
          w = w16[i]
          for cg in range(FGROUPS):
            sl = pl.ds(cg * 16, 16)
            rr[e, sl] = rr[e, sl] * w
        return 0
      lax.fori_loop(0, CHUNK // 16, scale_body, 0)

    def start_idx(j, b):
      pltpu.async_copy(pk_hbm.at[wid, j], pkb.at[b], isem.at[b])

    def wait_idx(j, b):
      pltpu.make_async_copy(pk_hbm.at[wid, j], pkb.at[b], isem.at[b]).wait()

    def start_gather(b):
      pltpu.async_copy(hp_hbm.at[pkb.at[b, 0]], rows.at[b], gsem.at[b])

    def wait_gather(b):
      pltpu.make_async_copy(
          hp_hbm.at[pkb.at[b, 0]], rows.at[b], gsem.at[b]).wait()

    def start_scatter(b):
      pltpu.async_copy(
          rows.at[b], acc_sh.at[pkb.at[b, 1]], ssem.at[b], add=True)

    def wait_scatter(b):
      pltpu.make_async_copy(
          rows.at[b], acc_sh.at[pkb.at[b, 1]], ssem.at[b]).wait()

    for b in range(3):
      start_idx(b, b)
    for b in range(3):
      wait_idx(b, b)
      start_gather(b)

    def group_body(g, _):
      for k in range(NR):
        j = g * NR + k
        kn = (k + 3) % NR
        @pl.when(j >= 2)
        def _():
          wait_scatter(kn)
        @pl.when(j + 3 < n_steps)
        def _():
          start_idx(j + 3, kn)
        wait_gather(k)
        scale(k)
        start_scatter(k)
        @pl.when(j + 3 < n_steps)
        def _():
          wait_idx(j + 3, kn)
          start_gather(kn)
      return 0

    lax.fori_loop(0, n_steps // NR, group_body, 0)
    for b in range(2):
      j = n_steps - 2 + b
      wait_scatter(j % NR)
    plsc.subcore_barrier()

    for k in range(rpt // CHUNK):
      r0 = base_row + k * CHUNK
      pltpu.async_copy(
          acc_sh.at[pl.ds(r0, CHUNK)], out_hbm.at[c, pl.ds(r0, CHUNK)], ssem.at[0])
    for k in range(rpt // CHUNK):
      r0 = base_row + k * CHUNK
      pltpu.make_async_copy(
          acc_sh.at[pl.ds(r0, CHUNK)], out_hbm.at[c, pl.ds(r0, CHUNK)], ssem.at[0]).wait()

  return pl.kernel(
      body,
      out_type=jax.ShapeDtypeStruct((NC, n_pad, H), jnp.float32),
      mesh=_mesh(),
      scratch_types=[
          pltpu.VMEM((NR, 3, CHUNK), jnp.int32),
          pltpu.VMEM((NR, CHUNK, H), jnp.float32),
          pltpu.VMEM_SHARED((n_pad, H), jnp.float32),
          pltpu.SemaphoreType.DMA((NR,)),
          pltpu.SemaphoreType.DMA((NR,)),
          pltpu.SemaphoreType.DMA((NR,)),
      ],
  )


def _make_deg(n_pad, e_pad):
  ept = e_pad // NW
  rpt = n_pad // NS
  n_steps = ept // CHUNK

  def body(dst_hbm, ew_hbm, out_hbm, dst2d, ew2d, zero_v, deg_sh, sem):
    c = lax.axis_index("c")
    s = lax.axis_index("s")
    wid = c * NS + s

    pltpu.sync_copy(dst_hbm.at[wid], dst2d)
    pltpu.sync_copy(ew_hbm.at[wid], ew2d)

    for g in range(CHUNK // 16):
      zero_v[pl.ds(g * 16, 16)] = jnp.zeros((16,), jnp.float32)
    base_row = s * rpt
    for k in range(rpt // CHUNK):
      pltpu.sync_copy(zero_v, deg_sh.at[pl.ds(base_row + k * CHUNK, CHUNK)])
    plsc.subcore_barrier()

    def fire(j, _):
      pltpu.async_copy(ew2d.at[j], deg_sh.at[dst2d.at[j]], sem, add=True)
      return 0
    lax.fori_loop(0, n_steps, fire, 0)

    def drain(j, _):
      pltpu.make_async_copy(ew2d.at[j], deg_sh.at[dst2d.at[j]], sem).wait()
      return 0
    lax.fori_loop(0, n_steps, drain, 0)
    plsc.subcore_barrier()

    for k in range(rpt // CHUNK):
      r0 = base_row + k * CHUNK
      pltpu.sync_copy(deg_sh.at[pl.ds(r0, CHUNK)], zero_v)
      pltpu.sync_copy(zero_v, out_hbm.at[c, pl.ds(r0, CHUNK)])

  return pl.kernel(
      body,
      out_type=jax.ShapeDtypeStruct((NC, n_pad), jnp.float32),
      mesh=_mesh(),
      scratch_types=[
          pltpu.VMEM((n_steps, CHUNK), jnp.int32),
          pltpu.VMEM((n_steps, CHUNK), jnp.float32),
          pltpu.VMEM((CHUNK,), jnp.float32),
          pltpu.VMEM_SHARED((n_pad,), jnp.float32),
          pltpu.SemaphoreType.DMA,
      ],
  )


_BM = 256


def _prep_body(p_ref, dis_ref, dinv_ref):
  d = p_ref[0] + p_ref[1] + 1.0
  dinv = 1.0 / d
  dinv_ref[...] = dinv
  dis_ref[...] = jnp.sqrt(dinv)


def _mm1_body(x_ref, w_ref, dis_ref, h_ref, hp_ref):
  h = jnp.dot(x_ref[...], w_ref[...], preferred_element_type=jnp.float32)
  h_ref[...] = h
  hp_ref[...] = h * dis_ref[...]


def _comb_mm_body(pa_ref, pb_ref, h_ref, dis_ref, dinv_ref, b_ref, w_ref,
                  out_ref, hn_ref, hpn_ref):
  o = jnp.maximum(
      dis_ref[...] * (pa_ref[...] + pb_ref[...])
      + dinv_ref[...] * h_ref[...] + b_ref[...], 0.0)
  out_ref[...] = o
  hn = jnp.dot(o, w_ref[...], preferred_element_type=jnp.float32)
  hn_ref[...] = hn
  hpn_ref[...] = hn * dis_ref[...]


def _comb_body(pa_ref, pb_ref, h_ref, dis_ref, dinv_ref, b_ref, out_ref):
  out_ref[...] = jnp.maximum(
      dis_ref[...] * (pa_ref[...] + pb_ref[...])
      + dinv_ref[...] * h_ref[...] + b_ref[...], 0.0)


def _row_specs(n_pad):
  blk = lambda i: (i, 0)
  full = lambda i: (0, 0)
  mat = pl.BlockSpec((_BM, H), blk)
  col = pl.BlockSpec((_BM, 1), blk)
  w = pl.BlockSpec((H, H), full)
  bias = pl.BlockSpec((1, H), full)
  return mat, col, w, bias, n_pad // _BM


def _mm1(n_pad, x, w1, dis_col):
  mat, col, w, _, g = _row_specs(n_pad)
  return pl.pallas_call(
      _mm1_body,
      grid=(g,),
      in_specs=[mat, w, col],
      out_specs=[mat, mat],
      out_shape=[jax.ShapeDtypeStruct((n_pad, H), jnp.float32)] * 2,
  )(x, w1, dis_col)


def _comb_mm(n_pad, pa, pb, h, dis_col, dinv_col, b_row, w_next):
  mat, col, w, bias, g = _row_specs(n_pad)
  return pl.pallas_call(
      _comb_mm_body,
      grid=(g,),
      in_specs=[mat, mat, mat, col, col, bias, w],
      out_specs=[mat, mat, mat],
      out_shape=[jax.ShapeDtypeStruct((n_pad, H), jnp.float32)] * 3,
  )(pa, pb, h, dis_col, dinv_col, b_row, w_next)


def _comb(n_pad, pa, pb, h, dis_col, dinv_col, b_row):
  mat, col, _, bias, g = _row_specs(n_pad)
  return pl.pallas_call(
      _comb_body,
      grid=(g,),
      in_specs=[mat, mat, mat, col, col, bias],
      out_specs=mat,
      out_shape=jax.ShapeDtypeStruct((n_pad, H), jnp.float32),
  )(pa, pb, h, dis_col, dinv_col, b_row)


def _prep(n_pad, deg_parts):
  p = deg_parts.reshape(NC, n_pad // H, H)
  spec = pl.BlockSpec(p.shape, lambda: (0, 0, 0))
  ospec = pl.BlockSpec(p.shape[1:], lambda: (0, 0))
  return pl.pallas_call(
      _prep_body,
      in_specs=[spec],
      out_specs=[ospec, ospec],
      out_shape=[jax.ShapeDtypeStruct(p.shape[1:], jnp.float32)] * 2,
  )(p)


def kernel(x, edge_index, edge_weights, W1, b1, W2, b2, W3, b3):
  n = x.shape[0]
  e = edge_index.shape[1]
  n_pad = ((n + NS * CHUNK - 1) // (NS * CHUNK)) * (NS * CHUNK)
  e_align = NW * CHUNK * NR
  e_pad = ((e + e_align - 1) // e_align) * e_align

  src = edge_index[0].astype(jnp.int32)
  dst = edge_index[1].astype(jnp.int32)
  ew = edge_weights.astype(jnp.float32)
  pad = e_pad - e
  if pad:
    pad_idx = jnp.arange(pad, dtype=jnp.int32) % n
    src = jnp.concatenate([src, pad_idx])
    dst = jnp.concatenate([dst, pad_idx])
    ew = jnp.concatenate([ew, jnp.zeros((pad,), jnp.float32)])
  x_p = jnp.pad(x, ((0, n_pad - n), (0, 0)))

  steps = e_pad // (NW * CHUNK)
  src = src.reshape(NW, steps, CHUNK)
  dst = dst.reshape(NW, steps, CHUNK)
  ew = ew.reshape(NW, steps, CHUNK)
  pk = jnp.stack([src, dst, lax.bitcast_convert_type(ew, jnp.int32)], axis=2)

  agg = _make_agg(n_pad, e_pad)
  deg_parts = _make_deg(n_pad, e_pad)(dst, ew)

  dis, dinv = _prep(n_pad, deg_parts)
  dis_col = dis.reshape(n_pad, 1)
  dinv_col = dinv.reshape(n_pad, 1)
  b1r, b2r, b3r = (b.reshape(1, H) for b in (b1, b2, b3))

  h1, hp1 = _mm1(n_pad, x_p, W1, dis_col)
  p1 = agg(hp1, pk)
  o1, h2, hp2 = _comb_mm(n_pad, p1[0], p1[1], h1, dis_col, dinv_col, b1r, W2)
  p2 = agg(hp2, pk)
  o2, h3, hp3 = _comb_mm(n_pad, p2[0], p2[1], h2, dis_col, dinv_col, b2r, W3)
  p3 = agg(hp3, pk)
  o3 = _comb(n_pad, p3[0], p3[1], h3, dis_col, dinv_col, b3r)

  return jnp.concatenate([o1[:n], o2[:n], o3[:n]], axis=1)


# speedup vs baseline: 1.0807x; 1.0015x over previous
import jax
import jax.numpy as jnp
from jax import lax
from jax.experimental import pallas as pl
from jax.experimental.pallas import tpu as pltpu
from jax.experimental.pallas import tpu_sc as plsc

H = 128
NC = 2
NS = 16
NW = NC * NS
CHUNK = 64
FGROUPS = H // 16

def _mesh():
  return plsc.VectorSubcoreMesh(
      core_axis_name="c", subcore_axis_name="s", num_cores=NC, num_subcores=NS)


def _zero_rows(rows_v, n_rows):
  def body(i, _):
    for cg in range(FGROUPS):
      rows_v[i, pl.ds(cg * 16, 16)] = jnp.zeros((16,), jnp.float32)
    return 0
  lax.fori_loop(0, n_rows, body, 0)


NR = 5


def _make_agg(n_pad, e_pad):
  ept = e_pad // NW
  rpt = n_pad // NS
  n_steps = ept // CHUNK

  def body(hp_hbm, pk_hbm, out_hbm, pkb, rows, acc_sh, isem, gsem, ssem):
    c = lax.axis_index("c")
    s = lax.axis_index("s")
    wid = c * NS + s

    _zero_rows(rows.at[0], CHUNK)
    base_row = s * rpt
    for k in range(rpt // CHUNK):
      pltpu.async_copy(
          rows.at[0], acc_sh.at[pl.ds(base_row + k * CHUNK, CHUNK)], ssem.at[0])
    for k in range(rpt // CHUNK):
      pltpu.make_async_copy(
          rows.at[0], acc_sh.at[pl.ds(base_row + k * CHUNK, CHUNK)], ssem.at[0]).wait()
    plsc.subcore_barrier()

    def scale(b):
      rr = rows.at[b]

      def scale_body(g, _):
        w16 = lax.bitcast_convert_type(pkb[b, 2, pl.ds(g * 16, 16)], jnp.float32)
        for i in range(16):
          e = g * 16 + i
          w = w16[i]
          for cg in range(FGROUPS):
            sl = pl.ds(cg * 16, 16)
            rr[e, sl] = rr[e, sl] * w
        return 0
      lax.fori_loop(0, CHUNK // 16, scale_body, 0)

    def start_idx(j, b):
      pltpu.async_copy(pk_hbm.at[wid, j], pkb.at[b], isem.at[b])

    def wait_idx(j, b):
      pltpu.make_async_copy(pk_hbm.at[wid, j], pkb.at[b], isem.at[b]).wait()

    def start_gather(b):
      pltpu.async_copy(hp_hbm.at[pkb.at[b, 0]], rows.at[b], gsem.at[b])

    def wait_gather(b):
      pltpu.make_async_copy(
          hp_hbm.at[pkb.at[b, 0]], rows.at[b], gsem.at[b]).wait()

    def start_scatter(b):
      pltpu.async_copy(
          rows.at[b], acc_sh.at[pkb.at[b, 1]], ssem.at[b], add=True)

    def wait_scatter(b):
      pltpu.make_async_copy(
          rows.at[b], acc_sh.at[pkb.at[b, 1]], ssem.at[b]).wait()

    for b in range(3):
      start_idx(b, b)
    for b in range(3):
      wait_idx(b, b)
      start_gather(b)

    def group_body(g, _):
      for k in range(NR):
        j = g * NR + k
        kn = (k + 3) % NR
        @pl.when(j >= 2)
        def _():
          wait_scatter(kn)
        @pl.when(j + 3 < n_steps)
        def _():
          start_idx(j + 3, kn)
        wait_gather(k)
        scale(k)
        start_scatter(k)
        @pl.when(j + 3 < n_steps)
        def _():
          wait_idx(j + 3, kn)
          start_gather(kn)
      return 0

    lax.fori_loop(0, n_steps // NR, group_body, 0)
    for b in range(2):
      j = n_steps - 2 + b
      wait_scatter(j % NR)
    plsc.subcore_barrier()

    for k in range(rpt // CHUNK):
      r0 = base_row + k * CHUNK
      pltpu.async_copy(
          acc_sh.at[pl.ds(r0, CHUNK)], out_hbm.at[c, pl.ds(r0, CHUNK)], ssem.at[0])
    for k in range(rpt // CHUNK):
      r0 = base_row + k * CHUNK
      pltpu.make_async_copy(
          acc_sh.at[pl.ds(r0, CHUNK)], out_hbm.at[c, pl.ds(r0, CHUNK)], ssem.at[0]).wait()

  return pl.kernel(
      body,
      out_type=jax.ShapeDtypeStruct((NC, n_pad, H), jnp.float32),
      mesh=_mesh(),
      scratch_types=[
          pltpu.VMEM((NR, 3, CHUNK), jnp.int32),
          pltpu.VMEM((NR, CHUNK, H), jnp.float32),
          pltpu.VMEM_SHARED((n_pad, H), jnp.float32),
          pltpu.SemaphoreType.DMA((NR,)),
          pltpu.SemaphoreType.DMA((NR,)),
          pltpu.SemaphoreType.DMA((NR,)),
      ],
  )


def _make_deg(n_pad, e_pad):
  ept = e_pad // NW
  rpt = n_pad // NS
  n_steps = ept // CHUNK

  def body(dst_hbm, ew_hbm, out_hbm, dst2d, ew2d, zero_v, deg_sh, sem):
    c = lax.axis_index("c")
    s = lax.axis_index("s")
    wid = c * NS + s

    pltpu.sync_copy(dst_hbm.at[wid], dst2d)
    pltpu.sync_copy(ew_hbm.at[wid], ew2d)

    for g in range(CHUNK // 16):
      zero_v[pl.ds(g * 16, 16)] = jnp.zeros((16,), jnp.float32)
    base_row = s * rpt
    for k in range(rpt // CHUNK):
      pltpu.sync_copy(zero_v, deg_sh.at[pl.ds(base_row + k * CHUNK, CHUNK)])
    plsc.subcore_barrier()

    def fire(j, _):
      pltpu.async_copy(ew2d.at[j], deg_sh.at[dst2d.at[j]], sem, add=True)
      return 0
    lax.fori_loop(0, n_steps, fire, 0)

    def drain(j, _):
      pltpu.make_async_copy(ew2d.at[j], deg_sh.at[dst2d.at[j]], sem).wait()
      return 0
    lax.fori_loop(0, n_steps, drain, 0)
    plsc.subcore_barrier()

    for k in range(rpt // CHUNK):
      r0 = base_row + k * CHUNK
      pltpu.sync_copy(deg_sh.at[pl.ds(r0, CHUNK)], zero_v)
      pltpu.sync_copy(zero_v, out_hbm.at[c, pl.ds(r0, CHUNK)])

  return pl.kernel(
      body,
      out_type=jax.ShapeDtypeStruct((NC, n_pad), jnp.float32),
      mesh=_mesh(),
      scratch_types=[
          pltpu.VMEM((n_steps, CHUNK), jnp.int32),
          pltpu.VMEM((n_steps, CHUNK), jnp.float32),
          pltpu.VMEM((CHUNK,), jnp.float32),
          pltpu.VMEM_SHARED((n_pad,), jnp.float32),
          pltpu.SemaphoreType.DMA,
      ],
  )


_BM = 256


def _prep_body(p_ref, dis_ref, dinv_ref):
  d = p_ref[0] + p_ref[1] + 1.0
  dinv = 1.0 / d
  dinv_ref[...] = dinv
  dis_ref[...] = jnp.sqrt(dinv)


def _mm1_body(x_ref, w_ref, dis_ref, h_ref, hp_ref):
  h = jnp.dot(x_ref[...], w_ref[...], preferred_element_type=jnp.float32)
  h_ref[...] = h
  hp_ref[...] = h * dis_ref[...]


def _comb_mm_body(pa_ref, pb_ref, h_ref, dis_ref, dinv_ref, b_ref, w_ref,
                  out_ref, hn_ref, hpn_ref):
  o = jnp.maximum(
      dis_ref[...] * (pa_ref[...] + pb_ref[...])
      + dinv_ref[...] * h_ref[...] + b_ref[...], 0.0)
  out_ref[...] = o
  hn = jnp.dot(o, w_ref[...], preferred_element_type=jnp.float32)
  hn_ref[...] = hn
  hpn_ref[...] = hn * dis_ref[...]


def _comb_body(pa_ref, pb_ref, h_ref, dis_ref, dinv_ref, b_ref, out_ref):
  out_ref[...] = jnp.maximum(
      dis_ref[...] * (pa_ref[...] + pb_ref[...])
      + dinv_ref[...] * h_ref[...] + b_ref[...], 0.0)


def _row_specs(n_pad):
  blk = lambda i: (i, 0)
  full = lambda i: (0, 0)
  mat = pl.BlockSpec((_BM, H), blk)
  col = pl.BlockSpec((_BM, 1), blk)
  w = pl.BlockSpec((H, H), full)
  bias = pl.BlockSpec((1, H), full)
  return mat, col, w, bias, n_pad // _BM


def _mm1(n_pad, x, w1, dis_col):
  mat, col, w, _, g = _row_specs(n_pad)
  return pl.pallas_call(
      _mm1_body,
      grid=(g,),
      in_specs=[mat, w, col],
      out_specs=[mat, mat],
      out_shape=[jax.ShapeDtypeStruct((n_pad, H), jnp.float32)] * 2,
  )(x, w1, dis_col)


def _comb_mm(n_pad, pa, pb, h, dis_col, dinv_col, b_row, w_next):
  mat, col, w, bias, g = _row_specs(n_pad)
  return pl.pallas_call(
      _comb_mm_body,
      grid=(g,),
      in_specs=[mat, mat, mat, col, col, bias, w],
      out_specs=[mat, mat, mat],
      out_shape=[jax.ShapeDtypeStruct((n_pad, H), jnp.float32)] * 3,
  )(pa, pb, h, dis_col, dinv_col, b_row, w_next)


def _comb(n_pad, pa, pb, h, dis_col, dinv_col, b_row):
  mat, col, _, bias, g = _row_specs(n_pad)
  return pl.pallas_call(
      _comb_body,
      grid=(g,),
      in_specs=[mat, mat, mat, col, col, bias],
      out_specs=mat,
      out_shape=jax.ShapeDtypeStruct((n_pad, H), jnp.float32),
  )(pa, pb, h, dis_col, dinv_col, b_row)


def _prep(n_pad, deg_parts):
  p = deg_parts.reshape(NC, n_pad // H, H)
  spec = pl.BlockSpec(p.shape, lambda: (0, 0, 0))
  ospec = pl.BlockSpec(p.shape[1:], lambda: (0, 0))
  return pl.pallas_call(
      _prep_body,
      in_specs=[spec],
      out_specs=[ospec, ospec],
      out_shape=[jax.ShapeDtypeStruct(p.shape[1:], jnp.float32)] * 2,
  )(p)


def kernel(x, edge_index, edge_weights, W1, b1, W2, b2, W3, b3):
  n = x.shape[0]
  e = edge_index.shape[1]
  n_pad = ((n + NS * CHUNK - 1) // (NS * CHUNK)) * (NS * CHUNK)
  e_align = NW * CHUNK * NR
  e_pad = ((e + e_align - 1) // e_align) * e_align

  src = edge_index[0].astype(jnp.int32)
  dst = edge_index[1].astype(jnp.int32)
  ew = edge_weights.astype(jnp.float32)
  pad = e_pad - e
  if pad:
    pad_idx = jnp.arange(pad, dtype=jnp.int32) % n
    src = jnp.concatenate([src, pad_idx])
    dst = jnp.concatenate([dst, pad_idx])
    ew = jnp.concatenate([ew, jnp.zeros((pad,), jnp.float32)])
  x_p = jnp.pad(x, ((0, n_pad - n), (0, 0)))

  steps = e_pad // (NW * CHUNK)
  src = src.reshape(NW, steps, CHUNK)
  dst = dst.reshape(NW, steps, CHUNK)
  ew = ew.reshape(NW, steps, CHUNK)
  pk = jnp.stack([src, dst, lax.bitcast_convert_type(ew, jnp.int32)], axis=2)

  agg = _make_agg(n_pad, e_pad)
  deg_parts = _make_deg(n_pad, e_pad)(dst, ew)

  dis, dinv = _prep(n_pad, deg_parts)
  dis_col = dis.reshape(n_pad, 1)
  dinv_col = dinv.reshape(n_pad, 1)
  b1r, b2r, b3r = (b.reshape(1, H) for b in (b1, b2, b3))

  h1, hp1 = _mm1(n_pad, x_p, W1, dis_col)
  p1 = agg(hp1, pk)
  o1, h2, hp2 = _comb_mm(n_pad, p1[0], p1[1], h1, dis_col, dinv_col, b1r, W2)
  p2 = agg(hp2, pk)
  o2, h3, hp3 = _comb_mm(n_pad, p2[0], p2[1], h2, dis_col, dinv_col, b2r, W3)
  p3 = agg(hp3, pk)
  o3 = _comb(n_pad, p3[0], p3[1], h3, dis_col, dinv_col, b3r)

  return jnp.concatenate([o1[:n], o2[:n], o3[:n]], axis=1)
